# baseline (speedup 1.0000x reference)
import functools

import jax
import jax.numpy as jnp
from jax import lax
from jax.experimental import pallas as pl
from jax.experimental.pallas import tpu as pltpu
from jax.experimental.pallas import tpu_sc as plsc

N, E, D, H = 10000, 320000, 128, 256
NC, NS, L = 2, 16, 16
NW = NC * NS
EPW = E // NW
KCH = 40
PAD = 2 * KCH
IB = 64
NCHUNK = -(-(EPW + PAD) // (IB * KCH)) * IB
CAP = NCHUNK * KCH
T0 = 5008
R0 = T0 // NS
R1 = (N - T0) // NS
ACC_ROWS = T0 + 1
DUMMY = T0

_mesh = plsc.VectorSubcoreMesh(core_axis_name="c", subcore_axis_name="s")
_sc_params = pltpu.CompilerParams(needs_layout_passes=False,
                                  use_tc_tiling_on_sc=False)


def _bin_body(ei_hbm, bsrc_hbm, bdst_hbm, cnt_hbm,
              ei_v, oa_s, oa_d, ob_s, ob_d, cnt_v):
    c = lax.axis_index("c")
    s = lax.axis_index("s")
    t = c * NS + s
    pltpu.sync_copy(ei_hbm.at[:, pl.ds(t * EPW, EPW)], ei_v)

    def fill(i, carry):
        z = jnp.zeros((L,), jnp.int32)
        dm = jnp.full((L,), DUMMY, jnp.int32)
        oa_s[pl.ds(i * L, L)] = z
        ob_s[pl.ds(i * L, L)] = z
        oa_d[pl.ds(i * L, L)] = dm
        ob_d[pl.ds(i * L, L)] = dm
        return carry
    lax.fori_loop(0, CAP // L, fill, jnp.int32(0))

    def body(i, carry):
        ca, cb = carry
        sv = ei_v[0, pl.ds(i * L, L)]
        dv = ei_v[1, pl.ds(i * L, L)]
        m_a = dv < T0
        m_b = jnp.logical_not(m_a)
        inc_a = jnp.where(m_a, 1, 0).astype(jnp.int32)
        cum_a = plsc.cumsum(inc_a)
        pos_a = ca + cum_a - 1
        pos_b = cb + (lax.iota(jnp.int32, L) - cum_a)
        plsc.store_scatter(oa_s, [pos_a], sv, mask=m_a)
        plsc.store_scatter(oa_d, [pos_a], dv, mask=m_a)
        plsc.store_scatter(ob_s, [pos_b], sv, mask=m_b)
        plsc.store_scatter(ob_d, [pos_b], dv - T0, mask=m_b)
        pa = jnp.max(cum_a)
        return ca + pa, cb + (L - pa)

    ca, cb = lax.fori_loop(0, EPW // L, body, (jnp.int32(0), jnp.int32(0)))

    ca_p = ((ca + PAD - 1) // PAD) * PAD
    cb_p = ((cb + PAD - 1) // PAD) * PAD
    lane = lax.iota(jnp.int32, L)
    cnt_v[...] = jnp.where(lane == 0, ca_p, jnp.where(lane == 1, cb_p, 0))
    pltpu.sync_copy(cnt_v, cnt_hbm.at[t])
    pltpu.sync_copy(oa_s, bsrc_hbm.at[0, t])
    pltpu.sync_copy(oa_d, bdst_hbm.at[0, t])
    pltpu.sync_copy(ob_s, bsrc_hbm.at[1, t])
    pltpu.sync_copy(ob_d, bdst_hbm.at[1, t])


_bin = pl.kernel(
    _bin_body,
    out_type=[jax.ShapeDtypeStruct((2, NW, CAP), jnp.int32),
              jax.ShapeDtypeStruct((2, NW, CAP), jnp.int32),
              jax.ShapeDtypeStruct((NW, L), jnp.int32)],
    mesh=_mesh,
    scratch_types=[pltpu.VMEM((2, EPW), jnp.int32),
                   pltpu.VMEM((CAP,), jnp.int32),
                   pltpu.VMEM((CAP,), jnp.int32),
                   pltpu.VMEM((CAP,), jnp.int32),
                   pltpu.VMEM((CAP,), jnp.int32),
                   pltpu.VMEM((L,), jnp.int32)],
    compiler_params=_sc_params,
)


def _seg_body(h_hbm, bsrc_hbm, bdst_hbm, cnt_hbm, agg_hbm,
              src_v, dst_v, rows_a, rows_b, cnt_v, acc,
              sem_a, sem_b, sem_c, sem_d):
    c = lax.axis_index("c")
    s = lax.axis_index("s")

    def zbody(i, carry):
        rows_a[i // (H // (2 * L)), pl.ds((i % (H // (2 * L))) * 2 * L, 2 * L)] = (
            jnp.zeros((2 * L,), jnp.bfloat16))
        return carry
    lax.fori_loop(0, KCH * (H // (2 * L)), zbody, jnp.int32(0))

    @pl.when(c == 0)
    def _():
        base = s * R0
        for j in range(R0 // KCH):
            pltpu.sync_copy(rows_a, acc.at[pl.ds(base + j * KCH, KCH)])
        if R0 % KCH:
            pltpu.sync_copy(rows_a.at[pl.ds(0, R0 % KCH)],
                            acc.at[pl.ds(base + (R0 // KCH) * KCH, R0 % KCH)])

    @pl.when(c == 1)
    def _():
        base = s * R1
        for j in range(R1 // KCH):
            pltpu.sync_copy(rows_a, acc.at[pl.ds(base + j * KCH, KCH)])
        if R1 % KCH:
            pltpu.sync_copy(rows_a.at[pl.ds(0, R1 % KCH)],
                            acc.at[pl.ds(base + (R1 // KCH) * KCH, R1 % KCH)])

    pltpu.sync_copy(cnt_hbm, cnt_v)
    plsc.subcore_barrier()

    for k in range(2):
        li = s * 2 + k
        csp = plsc.load_gather(
            cnt_v, [jnp.zeros((L,), jnp.int32) + li,
                    jnp.zeros((L,), jnp.int32) + c])
        nch = jnp.max(csp) // KCH

        @pl.when(nch > 0)
        def _(li=li, nch=nch):
            pltpu.sync_copy(bsrc_hbm.at[c, li, pl.ds(0, IB)], src_v)
            pltpu.sync_copy(bdst_hbm.at[c, li, pl.ds(0, IB)], dst_v)
            pltpu.async_copy(h_hbm.at[src_v.at[0]], rows_a, sem_a)
            npair = nch // 2

            def pbody(p, carry):
                i0 = (2 * p) % IB

                pltpu.async_copy(h_hbm.at[src_v.at[i0 + 1]], rows_b, sem_b)
                pltpu.make_async_copy(h_hbm.at[src_v.at[i0]],
                                      rows_a, sem_a).wait()
                pltpu.sync_copy(rows_a, acc.at[dst_v.at[i0]], add=True)

                @pl.when(jnp.logical_and(p + 1 < npair, i0 + 2 < IB))
                def _():
                    pltpu.async_copy(h_hbm.at[src_v.at[i0 + 2]],
                                     rows_a, sem_a)

                pltpu.make_async_copy(h_hbm.at[src_v.at[i0 + 1]],
                                      rows_b, sem_b).wait()
                pltpu.sync_copy(rows_b, acc.at[dst_v.at[i0 + 1]], add=True)

                @pl.when(jnp.logical_and(p + 1 < npair, i0 + 2 >= IB))
                def _():
                    pltpu.sync_copy(bsrc_hbm.at[c, li, pl.ds(2 * p + 2, IB)],
                                    src_v)
                    pltpu.sync_copy(bdst_hbm.at[c, li, pl.ds(2 * p + 2, IB)],
                                    dst_v)
                    pltpu.async_copy(h_hbm.at[src_v.at[0]], rows_a, sem_a)
                return carry
            lax.fori_loop(0, npair, pbody, jnp.int32(0))

    plsc.subcore_barrier()

    @pl.when(c == 0)
    def _():
        pltpu.sync_copy(acc.at[pl.ds(s * R0, R0)],
                        agg_hbm.at[pl.ds(s * R0, R0)])

    @pl.when(c == 1)
    def _():
        pltpu.sync_copy(acc.at[pl.ds(s * R1, R1)],
                        agg_hbm.at[pl.ds(T0 + s * R1, R1)])


_segsum = pl.kernel(
    _seg_body,
    out_type=jax.ShapeDtypeStruct((N, H), jnp.bfloat16),
    mesh=_mesh,
    scratch_types=[pltpu.VMEM((IB, KCH), jnp.int32),
                   pltpu.VMEM((IB, KCH), jnp.int32),
                   pltpu.VMEM((KCH, H), jnp.bfloat16),
                   pltpu.VMEM((KCH, H), jnp.bfloat16),
                   pltpu.VMEM((NW, L), jnp.int32),
                   pltpu.VMEM_SHARED((ACC_ROWS, H), jnp.bfloat16),
                   pltpu.SemaphoreType.DMA,
                   pltpu.SemaphoreType.DMA,
                   pltpu.SemaphoreType.DMA,
                   pltpu.SemaphoreType.DMA],
    compiler_params=_sc_params,
)


_BM = 2000


def _enc_body(x_ref, w_ref, b_ref, o_ref, o16_ref):
    o = (jnp.dot(x_ref[...], w_ref[...],
                 preferred_element_type=jnp.float32) + b_ref[...])
    o_ref[...] = o
    o16_ref[...] = o.astype(jnp.bfloat16)


def _encode(x, w, b):
    return pl.pallas_call(
        _enc_body,
        grid=(N // _BM,),
        in_specs=[pl.BlockSpec((_BM, D), lambda i: (i, 0)),
                  pl.BlockSpec((D, H), lambda i: (0, 0)),
                  pl.BlockSpec((1, H), lambda i: (0, 0))],
        out_specs=[pl.BlockSpec((_BM, H), lambda i: (i, 0)),
                   pl.BlockSpec((_BM, H), lambda i: (i, 0))],
        out_shape=[jax.ShapeDtypeStruct((N, H), jnp.float32),
                   jax.ShapeDtypeStruct((N, H), jnp.bfloat16)],
    )(x, w, b.reshape(1, H))


def _mlp_body(a_ref, h_ref, w1_ref, b1_ref, w2_ref, b2_ref, *o_refs, elu):
    z = a_ref[...].astype(jnp.float32) + h_ref[...]
    t = jnp.maximum(jnp.dot(z, w1_ref[...],
                            preferred_element_type=jnp.float32) + b1_ref[...],
                    0.0)
    o = jnp.dot(t, w2_ref[...],
                preferred_element_type=jnp.float32) + b2_ref[...]
    if elu:
        o = jnp.where(o > 0.0, o, jnp.exp(jnp.minimum(o, 0.0)) - 1.0)
        o_refs[0][...] = o
        o_refs[1][...] = o.astype(jnp.bfloat16)
    else:
        o_refs[0][...] = o


def _mlp(a, h, w1, b1, w2, b2, elu):
    if elu:
        out_specs = [pl.BlockSpec((_BM, H), lambda i: (i, 0)),
                     pl.BlockSpec((_BM, H), lambda i: (i, 0))]
        out_shape = [jax.ShapeDtypeStruct((N, H), jnp.float32),
                     jax.ShapeDtypeStruct((N, H), jnp.bfloat16)]
    else:
        out_specs = pl.BlockSpec((_BM, H), lambda i: (i, 0))
        out_shape = jax.ShapeDtypeStruct((N, H), jnp.float32)
    return pl.pallas_call(
        functools.partial(_mlp_body, elu=elu),
        grid=(N // _BM,),
        in_specs=[pl.BlockSpec((_BM, H), lambda i: (i, 0)),
                  pl.BlockSpec((_BM, H), lambda i: (i, 0)),
                  pl.BlockSpec((H, H), lambda i: (0, 0)),
                  pl.BlockSpec((1, H), lambda i: (0, 0)),
                  pl.BlockSpec((H, H), lambda i: (0, 0)),
                  pl.BlockSpec((1, H), lambda i: (0, 0))],
        out_specs=out_specs,
        out_shape=out_shape,
    )(a, h, w1, b1.reshape(1, H), w2, b2.reshape(1, H))


def kernel(x, edge_index, W_enc, b_enc, W1_0, b1_0, W2_0, b2_0,
           W1_1, b1_1, W2_1, b2_1):
    ei = edge_index.astype(jnp.int32)
    bsrc, bdst, counts = _bin(ei)
    bsrc = bsrc.reshape(2, NW, NCHUNK, KCH)
    bdst = bdst.reshape(2, NW, NCHUNK, KCH)
    h0, h0_16 = _encode(x, W_enc, b_enc)
    agg0 = _segsum(h0_16, bsrc, bdst, counts)
    h1, h1_16 = _mlp(agg0, h0, W1_0, b1_0, W2_0, b2_0, elu=True)
    agg1 = _segsum(h1_16, bsrc, bdst, counts)
    return _mlp(agg1, h1, W1_1, b1_1, W2_1, b2_1, elu=False)


# speedup vs baseline: 1.0089x; 1.0089x over previous
import functools

import jax
import jax.numpy as jnp
from jax import lax
from jax.experimental import pallas as pl
from jax.experimental.pallas import tpu as pltpu
from jax.experimental.pallas import tpu_sc as plsc

N, E, D, H = 10000, 320000, 128, 256
NC, NS, L = 2, 16, 16
NW = NC * NS
EPW = E // NW
KCH = 48
PAD = 2 * KCH
IB = 64
NCHUNK = -(-(EPW + PAD) // (IB * KCH)) * IB
CAP = NCHUNK * KCH
T0 = 5008
R0 = T0 // NS
R1 = (N - T0) // NS
ACC_ROWS = T0 + 1
DUMMY = T0

_mesh = plsc.VectorSubcoreMesh(core_axis_name="c", subcore_axis_name="s")
_sc_params = pltpu.CompilerParams(needs_layout_passes=False,
                                  use_tc_tiling_on_sc=False)


def _bin_body(ei_hbm, bsrc_hbm, bdst_hbm, cnt_hbm,
              ei_v, oa_s, oa_d, ob_s, ob_d, cnt_v):
    c = lax.axis_index("c")
    s = lax.axis_index("s")
    t = c * NS + s
    pltpu.sync_copy(ei_hbm.at[:, pl.ds(t * EPW, EPW)], ei_v)

    def fill(i, carry):
        z = jnp.zeros((L,), jnp.int32)
        dm = jnp.full((L,), DUMMY, jnp.int32)
        oa_s[pl.ds(i * L, L)] = z
        ob_s[pl.ds(i * L, L)] = z
        oa_d[pl.ds(i * L, L)] = dm
        ob_d[pl.ds(i * L, L)] = dm
        return carry
    lax.fori_loop(0, CAP // L, fill, jnp.int32(0))

    def body(i, carry):
        ca, cb = carry
        sv = ei_v[0, pl.ds(i * L, L)]
        dv = ei_v[1, pl.ds(i * L, L)]
        m_a = dv < T0
        m_b = jnp.logical_not(m_a)
        inc_a = jnp.where(m_a, 1, 0).astype(jnp.int32)
        cum_a = plsc.cumsum(inc_a)
        pos_a = ca + cum_a - 1
        pos_b = cb + (lax.iota(jnp.int32, L) - cum_a)
        plsc.store_scatter(oa_s, [pos_a], sv, mask=m_a)
        plsc.store_scatter(oa_d, [pos_a], dv, mask=m_a)
        plsc.store_scatter(ob_s, [pos_b], sv, mask=m_b)
        plsc.store_scatter(ob_d, [pos_b], dv - T0, mask=m_b)
        pa = jnp.max(cum_a)
        return ca + pa, cb + (L - pa)

    ca, cb = lax.fori_loop(0, EPW // L, body, (jnp.int32(0), jnp.int32(0)))

    ca_p = ((ca + PAD - 1) // PAD) * PAD
    cb_p = ((cb + PAD - 1) // PAD) * PAD
    lane = lax.iota(jnp.int32, L)
    cnt_v[...] = jnp.where(lane == 0, ca_p, jnp.where(lane == 1, cb_p, 0))
    pltpu.sync_copy(cnt_v, cnt_hbm.at[t])
    pltpu.sync_copy(oa_s, bsrc_hbm.at[0, t])
    pltpu.sync_copy(oa_d, bdst_hbm.at[0, t])
    pltpu.sync_copy(ob_s, bsrc_hbm.at[1, t])
    pltpu.sync_copy(ob_d, bdst_hbm.at[1, t])


_bin = pl.kernel(
    _bin_body,
    out_type=[jax.ShapeDtypeStruct((2, NW, CAP), jnp.int32),
              jax.ShapeDtypeStruct((2, NW, CAP), jnp.int32),
              jax.ShapeDtypeStruct((NW, L), jnp.int32)],
    mesh=_mesh,
    scratch_types=[pltpu.VMEM((2, EPW), jnp.int32),
                   pltpu.VMEM((CAP,), jnp.int32),
                   pltpu.VMEM((CAP,), jnp.int32),
                   pltpu.VMEM((CAP,), jnp.int32),
                   pltpu.VMEM((CAP,), jnp.int32),
                   pltpu.VMEM((L,), jnp.int32)],
    compiler_params=_sc_params,
)


def _seg_body(h_hbm, bsrc_hbm, bdst_hbm, cnt_hbm, agg_hbm,
              src_v, dst_v, rows_a, rows_b, cnt_v, acc, sem_a, sem_b):
    c = lax.axis_index("c")
    s = lax.axis_index("s")

    def zbody(i, carry):
        rows_a[i // (H // (2 * L)), pl.ds((i % (H // (2 * L))) * 2 * L, 2 * L)] = (
            jnp.zeros((2 * L,), jnp.bfloat16))
        return carry
    lax.fori_loop(0, KCH * (H // (2 * L)), zbody, jnp.int32(0))

    @pl.when(c == 0)
    def _():
        base = s * R0
        for j in range(R0 // KCH):
            pltpu.sync_copy(rows_a, acc.at[pl.ds(base + j * KCH, KCH)])
        if R0 % KCH:
            pltpu.sync_copy(rows_a.at[pl.ds(0, R0 % KCH)],
                            acc.at[pl.ds(base + (R0 // KCH) * KCH, R0 % KCH)])

    @pl.when(c == 1)
    def _():
        base = s * R1
        for j in range(R1 // KCH):
            pltpu.sync_copy(rows_a, acc.at[pl.ds(base + j * KCH, KCH)])
        if R1 % KCH:
            pltpu.sync_copy(rows_a.at[pl.ds(0, R1 % KCH)],
                            acc.at[pl.ds(base + (R1 // KCH) * KCH, R1 % KCH)])

    pltpu.sync_copy(cnt_hbm, cnt_v)
    plsc.subcore_barrier()

    for k in range(2):
        li = s * 2 + k
        csp = plsc.load_gather(
            cnt_v, [jnp.zeros((L,), jnp.int32) + li,
                    jnp.zeros((L,), jnp.int32) + c])
        nch = jnp.max(csp) // KCH

        @pl.when(nch > 0)
        def _(li=li, nch=nch):
            pltpu.sync_copy(bsrc_hbm.at[c, li, pl.ds(0, IB)], src_v)
            pltpu.sync_copy(bdst_hbm.at[c, li, pl.ds(0, IB)], dst_v)
            pltpu.async_copy(h_hbm.at[src_v.at[0]], rows_a, sem_a)
            npair = nch // 2

            def pbody(p, carry):
                i0 = (2 * p) % IB

                pltpu.async_copy(h_hbm.at[src_v.at[i0 + 1]], rows_b, sem_b)
                pltpu.make_async_copy(h_hbm.at[src_v.at[i0]],
                                      rows_a, sem_a).wait()
                pltpu.sync_copy(rows_a, acc.at[dst_v.at[i0]], add=True)

                @pl.when(jnp.logical_and(p + 1 < npair, i0 + 2 < IB))
                def _():
                    pltpu.async_copy(h_hbm.at[src_v.at[i0 + 2]],
                                     rows_a, sem_a)

                pltpu.make_async_copy(h_hbm.at[src_v.at[i0 + 1]],
                                      rows_b, sem_b).wait()
                pltpu.sync_copy(rows_b, acc.at[dst_v.at[i0 + 1]], add=True)

                @pl.when(jnp.logical_and(p + 1 < npair, i0 + 2 >= IB))
                def _():
                    pltpu.sync_copy(bsrc_hbm.at[c, li, pl.ds(2 * p + 2, IB)],
                                    src_v)
                    pltpu.sync_copy(bdst_hbm.at[c, li, pl.ds(2 * p + 2, IB)],
                                    dst_v)
                    pltpu.async_copy(h_hbm.at[src_v.at[0]], rows_a, sem_a)
                return carry
            lax.fori_loop(0, npair, pbody, jnp.int32(0))

    plsc.subcore_barrier()

    @pl.when(c == 0)
    def _():
        pltpu.sync_copy(acc.at[pl.ds(s * R0, R0)],
                        agg_hbm.at[pl.ds(s * R0, R0)])

    @pl.when(c == 1)
    def _():
        pltpu.sync_copy(acc.at[pl.ds(s * R1, R1)],
                        agg_hbm.at[pl.ds(T0 + s * R1, R1)])


_segsum = pl.kernel(
    _seg_body,
    out_type=jax.ShapeDtypeStruct((N, H), jnp.bfloat16),
    mesh=_mesh,
    scratch_types=[pltpu.VMEM((IB, KCH), jnp.int32),
                   pltpu.VMEM((IB, KCH), jnp.int32),
                   pltpu.VMEM((KCH, H), jnp.bfloat16),
                   pltpu.VMEM((KCH, H), jnp.bfloat16),
                   pltpu.VMEM((NW, L), jnp.int32),
                   pltpu.VMEM_SHARED((ACC_ROWS, H), jnp.bfloat16),
                   pltpu.SemaphoreType.DMA,
                   pltpu.SemaphoreType.DMA],
    compiler_params=_sc_params,
)


_BM = 2000


def _enc_body(x_ref, w_ref, b_ref, o_ref, o16_ref):
    o = (jnp.dot(x_ref[...], w_ref[...],
                 preferred_element_type=jnp.float32) + b_ref[...])
    o_ref[...] = o
    o16_ref[...] = o.astype(jnp.bfloat16)


def _encode(x, w, b):
    return pl.pallas_call(
        _enc_body,
        grid=(N // _BM,),
        in_specs=[pl.BlockSpec((_BM, D), lambda i: (i, 0)),
                  pl.BlockSpec((D, H), lambda i: (0, 0)),
                  pl.BlockSpec((1, H), lambda i: (0, 0))],
        out_specs=[pl.BlockSpec((_BM, H), lambda i: (i, 0)),
                   pl.BlockSpec((_BM, H), lambda i: (i, 0))],
        out_shape=[jax.ShapeDtypeStruct((N, H), jnp.float32),
                   jax.ShapeDtypeStruct((N, H), jnp.bfloat16)],
    )(x, w, b.reshape(1, H))


def _mlp_body(a_ref, h_ref, w1_ref, b1_ref, w2_ref, b2_ref, *o_refs, elu):
    z = a_ref[...].astype(jnp.float32) + h_ref[...]
    t = jnp.maximum(jnp.dot(z, w1_ref[...],
                            preferred_element_type=jnp.float32) + b1_ref[...],
                    0.0)
    o = jnp.dot(t, w2_ref[...],
                preferred_element_type=jnp.float32) + b2_ref[...]
    if elu:
        o = jnp.where(o > 0.0, o, jnp.exp(jnp.minimum(o, 0.0)) - 1.0)
        o_refs[0][...] = o
        o_refs[1][...] = o.astype(jnp.bfloat16)
    else:
        o_refs[0][...] = o


def _mlp(a, h, w1, b1, w2, b2, elu):
    if elu:
        out_specs = [pl.BlockSpec((_BM, H), lambda i: (i, 0)),
                     pl.BlockSpec((_BM, H), lambda i: (i, 0))]
        out_shape = [jax.ShapeDtypeStruct((N, H), jnp.float32),
                     jax.ShapeDtypeStruct((N, H), jnp.bfloat16)]
    else:
        out_specs = pl.BlockSpec((_BM, H), lambda i: (i, 0))
        out_shape = jax.ShapeDtypeStruct((N, H), jnp.float32)
    return pl.pallas_call(
        functools.partial(_mlp_body, elu=elu),
        grid=(N // _BM,),
        in_specs=[pl.BlockSpec((_BM, H), lambda i: (i, 0)),
                  pl.BlockSpec((_BM, H), lambda i: (i, 0)),
                  pl.BlockSpec((H, H), lambda i: (0, 0)),
                  pl.BlockSpec((1, H), lambda i: (0, 0)),
                  pl.BlockSpec((H, H), lambda i: (0, 0)),
                  pl.BlockSpec((1, H), lambda i: (0, 0))],
        out_specs=out_specs,
        out_shape=out_shape,
    )(a, h, w1, b1.reshape(1, H), w2, b2.reshape(1, H))


def kernel(x, edge_index, W_enc, b_enc, W1_0, b1_0, W2_0, b2_0,
           W1_1, b1_1, W2_1, b2_1):
    ei = edge_index.astype(jnp.int32)
    bsrc, bdst, counts = _bin(ei)
    bsrc = bsrc.reshape(2, NW, NCHUNK, KCH)
    bdst = bdst.reshape(2, NW, NCHUNK, KCH)
    h0, h0_16 = _encode(x, W_enc, b_enc)
    agg0 = _segsum(h0_16, bsrc, bdst, counts)
    h1, h1_16 = _mlp(agg0, h0, W1_0, b1_0, W2_0, b2_0, elu=True)
    agg1 = _segsum(h1_16, bsrc, bdst, counts)
    return _mlp(agg1, h1, W1_1, b1_1, W2_1, b2_1, elu=False)
